# Initial kernel scaffold; baseline (speedup 1.0000x reference)
import jax
import jax.numpy as jnp
from jax.experimental import pallas as pl


def kernel(x, W1, b1, W2, b2, up_w, up_b, e1w0, e1b0, e1w1, e1b1, e1w2, e1b2, e2w0, e2b0, e2w1, e2b1, e2w2, e2b2, e3w0, e3b0, e3w1, e3b1, e3w2, e3b2):
    raise NotImplementedError("write your pallas kernel here")


# speedup vs baseline: 8.0337x; 8.0337x over previous
import jax
import jax.numpy as jnp
from jax.experimental import pallas as pl

_dot = jnp.dot

_B = 128
_N = 128
_K = 16


def _topk_gather_mat(dist, k):
    n = dist.shape[0]
    neg = -dist
    iota = jax.lax.broadcasted_iota(jnp.int32, (n, n), 1)
    ohs = []
    for t in range(k + 1):
        m = jnp.max(neg, axis=1, keepdims=True)
        elig = neg == m
        idx = jnp.where(elig, iota, n)
        amin = jnp.min(idx, axis=1, keepdims=True)
        oh = iota == amin
        if t > 0:
            ohs.append(oh.astype(jnp.float32))
        neg = jnp.where(oh, -jnp.inf, neg)
    return jnp.concatenate(ohs, axis=0)


def _edge_conv(y, W0, b0, W1, b1, W2, b2):
    n, C = y.shape
    sq = jnp.sum(y * y, axis=1)
    inner = jax.lax.dot_general(y, y, (((1,), (1,)), ((), ())))
    dist = sq[:, None] - 2.0 * inner + sq[None, :]
    G = _topk_gather_mat(dist, _K)
    u = _dot(y, W0[:C]) + b0
    xj = _dot(G, y)
    dx = xj - jnp.concatenate([y] * _K, axis=0)
    f = jnp.concatenate([u] * _K, axis=0) + _dot(dx, W0[C:])
    f = jnp.maximum(f, 0.0)
    f = jnp.maximum(_dot(f, W1) + b1, 0.0)
    f = jnp.maximum(_dot(f, W2) + b2, 0.0)
    acc = f[0:n]
    for t in range(1, _K):
        acc = acc + f[t * n:(t + 1) * n]
    return acc * (1.0 / _K)


def _pre_kernel(x_ref, W1_ref, b1_ref, W2_ref, b2_ref, out_ref):
    x = x_ref[...]
    m = jnp.max(x, axis=0, keepdims=True)
    e = jnp.exp(x - m)
    s = jnp.sum(e, axis=0, keepdims=True)
    y = e / s
    y = _dot(y, W1_ref[...]) + b1_ref[...]
    y = _dot(y, W2_ref[...]) + b2_ref[...]
    out_ref[...] = y


def _main_kernel(y2_ref, upw_ref, upb_ref,
                 e1w0_ref, e1b0_ref, e1w1_ref, e1b1_ref, e1w2_ref, e1b2_ref,
                 e2w0_ref, e2b0_ref, e2w1_ref, e2b1_ref, e2w2_ref, e2b2_ref,
                 e3w0_ref, e3b0_ref, e3w1_ref, e3b1_ref, e3w2_ref, e3b2_ref,
                 out_ref):
    c = y2_ref[0]
    w = upw_ref[...]
    bconst = upb_ref[...]
    y = w * c + bconst
    y = _edge_conv(y, e1w0_ref[...], e1b0_ref[...], e1w1_ref[...],
                   e1b1_ref[...], e1w2_ref[...], e1b2_ref[...])
    y = _edge_conv(y, e2w0_ref[...], e2b0_ref[...], e2w1_ref[...],
                   e2b1_ref[...], e2w2_ref[...], e2b2_ref[...])
    y = _edge_conv(y, e3w0_ref[...], e3b0_ref[...], e3w1_ref[...],
                   e3b1_ref[...], e3w2_ref[...], e3b2_ref[...])
    out_ref[...] = y[None]


def kernel(x, W1, b1, W2, b2, up_w, up_b,
           e1w0, e1b0, e1w1, e1b1, e1w2, e1b2,
           e2w0, e2b0, e2w1, e2b1, e2w2, e2b2,
           e3w0, e3b0, e3w1, e3b1, e3w2, e3b2):
    B, ENC = x.shape
    n = up_w.shape[0]
    D = e3w2.shape[1]

    y2 = pl.pallas_call(
        _pre_kernel,
        out_shape=jax.ShapeDtypeStruct((B, W2.shape[1]), jnp.float32),
    )(x, W1, b1[None, :], W2, b2[None, :])

    rep2 = lambda a: pl.BlockSpec(a.shape, lambda b: (0, 0))
    weights = [e1w0, e1b0[None, :], e1w1, e1b1[None, :], e1w2, e1b2[None, :],
               e2w0, e2b0[None, :], e2w1, e2b1[None, :], e2w2, e2b2[None, :],
               e3w0, e3b0[None, :], e3w1, e3b1[None, :], e3w2, e3b2[None, :]]

    out = pl.pallas_call(
        _main_kernel,
        grid=(B,),
        in_specs=[
            pl.BlockSpec((1, 1, W2.shape[1]), lambda b: (b, 0, 0)),
            rep2(up_w[:, None]),
            rep2(up_b[:, None]),
        ] + [rep2(a) for a in weights],
        out_specs=pl.BlockSpec((1, n, D), lambda b: (b, 0, 0)),
        out_shape=jax.ShapeDtypeStruct((B, n, D), jnp.float32),
    )(y2[:, None, :], up_w[:, None], up_b[:, None], *weights)
    return out
